# Initial kernel scaffold; baseline (speedup 1.0000x reference)
import jax
import jax.numpy as jnp
from jax.experimental import pallas as pl


def kernel(features_list, norm, node_embeddings, linear_w, linear_b, out_w, out_b, g, r, num_nodes, num_relations, args_dataset):
    raise NotImplementedError("write your pallas kernel here")


# speedup vs baseline: 7.3943x; 7.3943x over previous
import functools
import math

import jax
import jax.numpy as jnp
from jax import lax
from jax.experimental import pallas as pl
from jax.experimental.pallas import tpu as pltpu
from jax.experimental.pallas import tpu_sc as plsc

N = 10000
E = 320000
RB = 4
R8 = 8
H = 128
HH = 64
RN = R8 * N
RNP = RN + 128
E2 = 2 * E
PAD2E = 655360
ROWS2E = PAD2E // 128
PADE = 327680
ROWSE = PADE // 128
LAM1 = 20.0
LAM2 = 30.0
C1 = 1.0 / (1.0 + LAM1)
C2 = LAM1 / (1.0 + LAM1)
NC = 2
NS = 16

_MESH = dict(core_axis_name="c", subcore_axis_name="s", num_cores=NC,
             num_subcores=NS)


def _f32(x):
    return x.astype(jnp.float32)


def _k1_body(idx_hbm, zeros_hbm, out_hbm, acc, ibuf, ones, obuf, sem):
    c = lax.axis_index("c")
    t = lax.axis_index("s")
    pltpu.sync_copy(zeros_hbm.at[t], acc.at[pl.ds(t * 5008, 5008)])
    for k in range(8):
        ones[pl.ds(16 * k, 16)] = jnp.full((16,), 1.0, jnp.float32)
    plsc.subcore_barrier()

    def blk(bi, carry):
        r0 = t * 320 + bi * 16
        pltpu.sync_copy(idx_hbm.at[c, pl.ds(r0, 16), :], ibuf)
        for b in range(16):
            pltpu.async_copy(ones, acc.at[ibuf.at[b]], sem, add=True)
        for b in range(16):
            pltpu.make_async_copy(ones, acc.at[ibuf.at[b]], sem).wait()
        return carry

    lax.fori_loop(0, 20, blk, 0)
    plsc.subcore_barrier()
    pltpu.sync_copy(acc.at[pl.ds(t * 5000, 5000)], obuf)
    pltpu.sync_copy(obuf, out_hbm.at[c, pl.ds(t * 5000, 5000)])


def _k1_call(idx2, zk1):
    kfn = pl.kernel(
        _k1_body,
        out_type=jax.ShapeDtypeStruct((2, RN), jnp.float32),
        mesh=plsc.VectorSubcoreMesh(**_MESH),
        compiler_params=pltpu.CompilerParams(use_tc_tiling_on_sc=False, needs_layout_passes=False),
        scratch_types=[
            pltpu.VMEM_SHARED((RNP,), jnp.float32),
            pltpu.VMEM((16, 128), jnp.int32),
            pltpu.VMEM((128,), jnp.float32),
            pltpu.VMEM((5000,), jnp.float32),
            pltpu.SemaphoreType.DMA,
        ],
    )
    return kfn(idx2, zk1)


def _k2_body(cnt_ref, isr_ref, isc_ref, inv_ref):
    rs = cnt_ref[0]
    dc = cnt_ref[1]
    isr_ref[...] = jnp.where(rs > 0, 1.0 / jnp.sqrt(jnp.maximum(rs, 1e-12)),
                             0.0)
    isc_ref[...] = jnp.where(dc > 0, 1.0 / jnp.sqrt(jnp.maximum(dc, 1e-12)),
                             0.0)
    inv_ref[...] = 1.0 / jnp.maximum(rs, 1.0)


def _k2_call(counts):
    out = jax.ShapeDtypeStruct((RN,), jnp.float32)
    return pl.pallas_call(
        _k2_body,
        out_shape=[out, out, out],
    )(counts)


def _k3_body(idx_hbm, isr_hbm, isc_hbm, inv_hbm, vals_hbm, vh_hbm,
             t_isr, t_isc, t_inv, ribuf, cibuf, g1, g2, g3, vhbuf, sem):
    c = lax.axis_index("c")
    t = lax.axis_index("s")
    sl = pl.ds(t * 5008, 5008)
    pltpu.sync_copy(isr_hbm.at[sl], t_isr.at[sl])
    pltpu.sync_copy(isc_hbm.at[sl], t_isc.at[sl])
    pltpu.sync_copy(inv_hbm.at[sl], t_inv.at[sl])
    plsc.subcore_barrier()
    wid = c * NS + t

    def row(i, carry):
        r = wid * 160 + i
        pltpu.sync_copy(idx_hbm.at[0, r, :], ribuf)
        pltpu.sync_copy(idx_hbm.at[1, r, :], cibuf)
        pltpu.async_copy(t_isr.at[ribuf], g1, sem)
        pltpu.make_async_copy(t_isr.at[ribuf], g1, sem).wait()
        pltpu.async_copy(t_isc.at[cibuf], g2, sem)
        pltpu.make_async_copy(t_isc.at[cibuf], g2, sem).wait()
        pltpu.async_copy(t_inv.at[ribuf], g3, sem)
        pltpu.make_async_copy(t_inv.at[ribuf], g3, sem).wait()
        for k in range(8):
            s16 = pl.ds(16 * k, 16)
            vhbuf[s16] = g1[s16] * g2[s16]
        pltpu.sync_copy(vhbuf, vh_hbm.at[r])
        pltpu.sync_copy(g3, vals_hbm.at[r])
        return carry

    lax.fori_loop(0, 160, row, 0)


def _k3_call(idx2, isr_t, isc_t, inv_t):
    out = jax.ShapeDtypeStruct((ROWS2E, 128), jnp.float32)
    kfn = pl.kernel(
        _k3_body,
        out_type=[out, out],
        mesh=plsc.VectorSubcoreMesh(**_MESH),
        compiler_params=pltpu.CompilerParams(use_tc_tiling_on_sc=False, needs_layout_passes=False),
        scratch_types=[
            pltpu.VMEM_SHARED((RNP,), jnp.float32),
            pltpu.VMEM_SHARED((RNP,), jnp.float32),
            pltpu.VMEM_SHARED((RNP,), jnp.float32),
            pltpu.VMEM((128,), jnp.int32),
            pltpu.VMEM((128,), jnp.int32),
            pltpu.VMEM((128,), jnp.float32),
            pltpu.VMEM((128,), jnp.float32),
            pltpu.VMEM((128,), jnp.float32),
            pltpu.VMEM((128,), jnp.float32),
            pltpu.SemaphoreType.DMA,
        ],
    )
    return kfn(idx2, isr_t, isc_t, inv_t)


def _k0_body(ne_ref, w_ref, b_ref, out_ref):
    hb = jnp.dot(ne_ref[...], w_ref[...],
                 preferred_element_type=jnp.float32) + b_ref[...]
    m = jnp.mean(hb, axis=1, keepdims=True)
    d = jnp.sqrt(jnp.sum((hb - m) * (hb - m), axis=1, keepdims=True)
                 / (H - 1))
    o = (hb - m) / d
    out_ref[...] = jnp.where(jnp.isnan(o), 0.0, o)


def _k0_call(ne, w, b):
    return pl.pallas_call(
        _k0_body,
        grid=(10,),
        in_specs=[
            pl.BlockSpec((1000, 128), lambda i: (i, 0)),
            pl.BlockSpec((128, 128), lambda i: (0, 0)),
            pl.BlockSpec((1, 128), lambda i: (0, 0)),
        ],
        out_specs=pl.BlockSpec((1000, 128), lambda i: (i, 0)),
        out_shape=jax.ShapeDtypeStruct((N, H), jnp.float32),
    )(ne, w, b)


def _k4_body(meta_hbm, xs2_hbm, tv_hbm, mbuf, gx, acc, gsem):
    c = lax.axis_index("c")
    t = lax.axis_index("s")
    wid = c * NS + t
    t0 = wid * 80
    for r in range(8):
        acc[r, :] = jnp.zeros((16,), jnp.float32)

    def issue(b, rr):
        pltpu.sync_copy(meta_hbm.at[rr], mbuf.at[b])
        for q in range(4):
            pltpu.async_copy(xs2_hbm.at[mbuf.at[b, q]], gx.at[b, q], gsem)

    def drain(b):
        for q in range(4):
            pltpu.make_async_copy(xs2_hbm.at[mbuf.at[b, q]], gx.at[b, q],
                                  gsem).wait()

    issue(0, t0)
    issue(1, t0 + 1)

    def pair(ii, carry):
        for b in range(2):
            drain(b)
            _tv_edges(b, mbuf, gx, acc)

            @pl.when(ii < 39)
            def _():
                issue(b, t0 + 2 * ii + b + 2)

        return carry

    lax.fori_loop(0, 40, pair, 0)
    pltpu.sync_copy(acc, tv_hbm.at[wid])


def _tv_edges(b, mbuf, gx, acc):
    def grp(jj, cy):
        base = 16 * jj
        bs = pl.ds(base, 16)
        p16 = mbuf[b, 4, bs]
        vf16 = plsc.bitcast(mbuf[b, 5, bs], jnp.float32)
        vm16 = plsc.bitcast(mbuf[b, 6, bs], jnp.float32)
        for l in range(16):
            j = base + l
            pv = gx[b, 0, j, pl.ds(0, 16)] * gx[b, 2, j, pl.ds(0, 16)]
            for k in range(1, 4):
                s16 = pl.ds(16 * k, 16)
                pv = pv + gx[b, 0, j, s16] * gx[b, 2, j, s16]
            for k in range(4):
                s16 = pl.ds(16 * k, 16)
                pv = pv + gx[b, 1, j, s16] * gx[b, 3, j, s16]
            pj = p16[l]
            acc[pj, :] = acc[pj, :] + vf16[l] * pv
            pj4 = pj + 4
            acc[pj4, :] = acc[pj4, :] + vm16[l] * pv
        return cy

    lax.fori_loop(0, 8, grp, 0)


def _k4_call(meta4, xs2):
    kfn = pl.kernel(
        _k4_body,
        out_type=jax.ShapeDtypeStruct((32, 8, 16), jnp.float32),
        mesh=plsc.VectorSubcoreMesh(**_MESH),
        compiler_params=pltpu.CompilerParams(use_tc_tiling_on_sc=False, needs_layout_passes=False),
        scratch_types=[
            pltpu.VMEM((2, 8, 128), jnp.int32),
            pltpu.VMEM((2, 4, 128, HH), jnp.float32),
            pltpu.VMEM((8, 16), jnp.float32),
            pltpu.SemaphoreType.DMA,
        ],
    )
    return kfn(meta4, xs2)


def _k6_body(meta_hbm, u16_hbm, xs2_hbm, zeros_hbm, xsn_hbm,
             accum, mbuf, grows, stage, wbuf, ubuf, gsem, ssem):
    c = lax.axis_index("c")
    t = lax.axis_index("s")
    pltpu.sync_copy(zeros_hbm.at[pl.ds(t * 625, 625), :],
                    accum.at[pl.ds(t * 625, 625), :])
    pltpu.sync_copy(u16_hbm, ubuf)
    plsc.subcore_barrier()
    coff = c * N
    t0 = t * 320

    def load_meta_and_gather(m, b, rr):
        pltpu.sync_copy(meta_hbm.at[c, rr], mbuf.at[m])
        pltpu.async_copy(xs2_hbm.at[mbuf.at[m, 0]], grows.at[b], gsem)

    load_meta_and_gather(0, 0, t0)
    load_meta_and_gather(1, 1, t0 + 1)

    def quad(ii, carry):
        for q in range(4):
            b = q % 2
            m = q
            i = 4 * ii + q
            r = t0 + i
            pltpu.make_async_copy(xs2_hbm.at[mbuf.at[m, 0]], grows.at[b],
                                  gsem).wait()

            @pl.when(i >= 2)
            def _():
                pltpu.make_async_copy(stage.at[b],
                                      accum.at[mbuf.at[m, 1]], ssem).wait()

            for k in range(8):
                s16 = pl.ds(16 * k, 16)
                pk = mbuf[m, 2, s16]
                vk = lax.bitcast_convert_type(mbuf[m, 3, s16], jnp.float32)
                wbuf[s16] = plsc.load_gather(ubuf, [pk]) * vk

            def grp(jj, cy):
                base = 16 * jj
                w16 = wbuf[pl.ds(base, 16)]
                for l in range(16):
                    j = base + l
                    wj = w16[l]
                    for k in range(4):
                        s16 = pl.ds(16 * k, 16)
                        stage[b, j, s16] = grows[b, j, s16] * wj
                return cy

            lax.fori_loop(0, 8, grp, 0)
            pltpu.async_copy(stage.at[b], accum.at[mbuf.at[m, 1]], ssem,
                             add=True)

            @pl.when(i + 2 < 320)
            def _():
                load_meta_and_gather((m + 2) % 4, b, r + 2)

        return carry

    lax.fori_loop(0, 80, quad, 0)
    for q in range(2, 4):
        pltpu.make_async_copy(stage.at[q % 2], accum.at[mbuf.at[q, 1]],
                              ssem).wait()
    plsc.subcore_barrier()

    def cblk(qq, carry):
        r0 = t * 625 + qq * 25
        g0 = coff + r0
        pltpu.sync_copy(accum.at[pl.ds(r0, 25), :], grows.at[0, pl.ds(0, 25)])
        pltpu.sync_copy(xs2_hbm.at[pl.ds(g0, 25), :],
                        grows.at[1, pl.ds(0, 25)])
        for i in range(25):
            for k in range(4):
                s16 = pl.ds(16 * k, 16)
                grows[1, i, s16] = (C1 * grows[1, i, s16]
                                    + C2 * grows[0, i, s16])
        pltpu.sync_copy(grows.at[1, pl.ds(0, 25)], xsn_hbm.at[pl.ds(g0, 25), :])
        return carry

    lax.fori_loop(0, 25, cblk, 0)


def _k6_call(meta6, u16, xs2, zk6):
    kfn = pl.kernel(
        _k6_body,
        out_type=jax.ShapeDtypeStruct((2 * N, HH), jnp.float32),
        mesh=plsc.VectorSubcoreMesh(**_MESH),
        compiler_params=pltpu.CompilerParams(use_tc_tiling_on_sc=False, needs_layout_passes=False),
        scratch_types=[
            pltpu.VMEM_SHARED((N, HH), jnp.float32),
            pltpu.VMEM((4, 4, 128), jnp.int32),
            pltpu.VMEM((2, 128, HH), jnp.float32),
            pltpu.VMEM((2, 128, HH), jnp.float32),
            pltpu.VMEM((128,), jnp.float32),
            pltpu.VMEM((16,), jnp.float32),
            pltpu.SemaphoreType.DMA,
            pltpu.SemaphoreType.DMA,
        ],
    )
    return kfn(meta6, u16, xs2, zk6)


def _k5_body(tv_ref, xs_ref, u_ref, uo_ref):
    dots = jnp.sum(tv_ref[...], axis=1)
    nsq = jnp.sum(xs_ref[...] * xs_ref[...])
    w = jnp.broadcast_to(((nsq - dots) / N)[:, None], (R8, 128))
    l1 = jnp.sum(jnp.abs(w[:, :1]))
    fi = l1 + 2.0 * LAM2 / LAM1
    u = u_ref[...]

    def inner(tt, uu):
        t_f = (tt + 1).astype(jnp.float32)
        T_t = jnp.sqrt(2.0 * math.log(R8) / (t_f * fi * fi))
        f_de = (2.0 * LAM2 / LAM1) * uu + w
        u_ta = uu * jnp.exp(-T_t * f_de)
        return u_ta / jnp.sum(u_ta[:, :1])

    uo_ref[...] = lax.fori_loop(0, 10, inner, u)


def _k5_call(tvr, xs2, u):
    return pl.pallas_call(
        _k5_body,
        out_shape=jax.ShapeDtypeStruct((R8, 128), jnp.float32),
    )(tvr, xs2, u)


def _k7_body(xa_ref, xb_ref, wa_ref, wb_ref, b_ref, o_ref):
    o_ref[...] = (jnp.dot(xa_ref[...], wa_ref[...],
                          preferred_element_type=jnp.float32)
                  + jnp.dot(xb_ref[...], wb_ref[...],
                            preferred_element_type=jnp.float32)
                  + b_ref[...])


def _k7_call(xs2, wa, wb, b):
    return pl.pallas_call(
        _k7_body,
        grid=(10,),
        in_specs=[
            pl.BlockSpec((1000, HH), lambda i: (i, 0)),
            pl.BlockSpec((1000, HH), lambda i: (i + 10, 0)),
            pl.BlockSpec((HH, 128), lambda i: (0, 0)),
            pl.BlockSpec((HH, 128), lambda i: (0, 0)),
            pl.BlockSpec((1, 128), lambda i: (0, 0)),
        ],
        out_specs=pl.BlockSpec((1000, 128), lambda i: (i, 0)),
        out_shape=jax.ShapeDtypeStruct((N, 128), jnp.float32),
    )(xs2, xs2, wa, wb, b)


def kernel(features_list, norm, node_embeddings, linear_w, linear_b, out_w,
           out_b, g, r, num_nodes, num_relations, args_dataset):
    del features_list, norm, r, num_nodes, num_relations, args_dataset
    g = g.astype(jnp.int32)
    s = g[:, 0]
    p = jnp.remainder(g[:, 1], RB)
    o = g[:, 2]
    s_all = jnp.concatenate([s, o])
    p_all = jnp.concatenate([p, p + RB])
    o_all = jnp.concatenate([o, s])
    rows = p_all * N + s_all
    colseg = p_all * N + o_all

    pad2 = PAD2E - E2
    padfill = jnp.full((pad2,), RN, jnp.int32)
    idx2 = jnp.stack([jnp.concatenate([rows, padfill]),
                      jnp.concatenate([colseg, padfill])]).reshape(
                          2, ROWS2E, 128)
    zk1 = jnp.zeros((16, 5008), jnp.float32)
    counts = _k1_call(idx2, zk1)

    isr2, isc2, inv2 = _k2_call(counts)
    zpad = jnp.zeros((128,), jnp.float32)
    isr_t = jnp.concatenate([isr2, zpad])
    isc_t = jnp.concatenate([isc2, zpad])
    inv_t = jnp.concatenate([inv2, zpad])

    vals2d, vh2d = _k3_call(idx2, isr_t, isc_t, inv_t)

    vh_flat = vh2d.reshape(PAD2E)
    vhf = vh_flat[:E]
    vhm = vh_flat[E:E2]
    padE = PADE - E
    zi = jnp.zeros((padE,), jnp.int32)
    zf = jnp.zeros((padE,), jnp.float32)

    def padi(x):
        return jnp.concatenate([x, zi])

    def padf(x):
        return lax.bitcast_convert_type(jnp.concatenate([x, zf]), jnp.int32)

    meta4 = jnp.stack([padi(s), padi(s + N), padi(o), padi(o + N), padi(p),
                       padf(vhf), padf(vhm), padi(jnp.zeros((E,), jnp.int32))],
                      axis=0)
    meta4 = meta4.reshape(8, ROWSE, 128).transpose(1, 0, 2)

    zi2 = jnp.zeros((pad2,), jnp.int32)

    def padi2(x):
        return jnp.concatenate([x, zi2])

    vals_bits = lax.bitcast_convert_type(vals2d.reshape(PAD2E), jnp.int32)
    meta6 = jnp.stack([
        jnp.stack([padi2(o_all), padi2(s_all), padi2(p_all), vals_bits]),
        jnp.stack([padi2(o_all + N), padi2(s_all), padi2(p_all), vals_bits]),
    ])
    meta6 = meta6.reshape(2, 4, ROWS2E, 128).transpose(0, 2, 1, 3)

    xn = _k0_call(node_embeddings, linear_w, linear_b.reshape(1, H))
    xs2 = jnp.concatenate([xn[:, :HH], xn[:, HH:]], axis=0)

    zk6 = jnp.zeros((N, HH), jnp.float32)
    u = jnp.full((R8, 128), 1.0 / R8, jnp.float32)
    for _ in range(2):
        tvt = _k4_call(meta4, xs2)
        tvr = tvt.transpose(1, 0, 2).reshape(R8, 512)
        u = _k5_call(tvr, xs2, u)
        u16 = jnp.concatenate([u[:, 0], jnp.zeros((8,), jnp.float32)])
        xs2 = _k6_call(meta6, u16, xs2, zk6)

    out_wp = jnp.pad(out_w, ((0, 0), (0, 112)))
    logits_full = _k7_call(xs2, out_wp[:HH], out_wp[HH:],
                           jnp.pad(out_b, (0, 112)).reshape(1, 128))
    logits = logits_full[:, :16]
    embeddings = jnp.concatenate([xs2[:N], xs2[N:]], axis=1)
    return logits, embeddings, u[:, :1]


# speedup vs baseline: 7.8939x; 1.0676x over previous
import functools
import math

import jax
import jax.numpy as jnp
from jax import lax
from jax.experimental import pallas as pl
from jax.experimental.pallas import tpu as pltpu
from jax.experimental.pallas import tpu_sc as plsc

N = 10000
E = 320000
RB = 4
R8 = 8
H = 128
HH = 64
RN = R8 * N
RNP = RN + 128
E2 = 2 * E
PAD2E = 655360
ROWS2E = PAD2E // 128
PADE = 327680
ROWSE = PADE // 128
LAM1 = 20.0
LAM2 = 30.0
C1 = 1.0 / (1.0 + LAM1)
C2 = LAM1 / (1.0 + LAM1)
NC = 2
NS = 16

_MESH = dict(core_axis_name="c", subcore_axis_name="s", num_cores=NC,
             num_subcores=NS)


def _f32(x):
    return x.astype(jnp.float32)


def _k1_body(idx_hbm, zeros_hbm, out_hbm, acc, ibuf, ones, obuf, sem):
    c = lax.axis_index("c")
    t = lax.axis_index("s")
    pltpu.sync_copy(zeros_hbm.at[t], acc.at[pl.ds(t * 5008, 5008)])
    for k in range(8):
        ones[pl.ds(16 * k, 16)] = jnp.full((16,), 1.0, jnp.float32)
    plsc.subcore_barrier()

    def blk(bi, carry):
        r0 = t * 320 + bi * 16
        pltpu.sync_copy(idx_hbm.at[c, pl.ds(r0, 16), :], ibuf)
        for b in range(16):
            pltpu.async_copy(ones, acc.at[ibuf.at[b]], sem, add=True)
        for b in range(16):
            pltpu.make_async_copy(ones, acc.at[ibuf.at[b]], sem).wait()
        return carry

    lax.fori_loop(0, 20, blk, 0)
    plsc.subcore_barrier()
    pltpu.sync_copy(acc.at[pl.ds(t * 5000, 5000)], obuf)
    pltpu.sync_copy(obuf, out_hbm.at[c, pl.ds(t * 5000, 5000)])


def _k1_call(idx2, zk1):
    kfn = pl.kernel(
        _k1_body,
        out_type=jax.ShapeDtypeStruct((2, RN), jnp.float32),
        mesh=plsc.VectorSubcoreMesh(**_MESH),
        compiler_params=pltpu.CompilerParams(use_tc_tiling_on_sc=False, needs_layout_passes=False),
        scratch_types=[
            pltpu.VMEM_SHARED((RNP,), jnp.float32),
            pltpu.VMEM((16, 128), jnp.int32),
            pltpu.VMEM((128,), jnp.float32),
            pltpu.VMEM((5000,), jnp.float32),
            pltpu.SemaphoreType.DMA,
        ],
    )
    return kfn(idx2, zk1)


def _k2_body(cnt_ref, isr_ref, isc_ref, inv_ref):
    rs = cnt_ref[0]
    dc = cnt_ref[1]
    isr_ref[...] = jnp.where(rs > 0, 1.0 / jnp.sqrt(jnp.maximum(rs, 1e-12)),
                             0.0)
    isc_ref[...] = jnp.where(dc > 0, 1.0 / jnp.sqrt(jnp.maximum(dc, 1e-12)),
                             0.0)
    inv_ref[...] = 1.0 / jnp.maximum(rs, 1.0)


def _k2_call(counts):
    out = jax.ShapeDtypeStruct((RN,), jnp.float32)
    return pl.pallas_call(
        _k2_body,
        out_shape=[out, out, out],
    )(counts)


def _k3_body(idx_hbm, isr_hbm, isc_hbm, inv_hbm, vals_hbm, vh_hbm,
             tbl, vhacc, ridx, ibuf, obuf):
    c = lax.axis_index("c")
    t = lax.axis_index("s")
    wid = c * NS + t
    r0 = wid * 160
    pltpu.sync_copy(idx_hbm.at[0, pl.ds(r0, 160), :], ridx)

    pltpu.sync_copy(isr_hbm, tbl)

    def p1(z, carry):
        for q in range(8):
            lz = z * 8 + q
            for k in range(8):
                s16 = pl.ds(16 * k, 16)
                vhacc[lz, s16] = plsc.load_gather(tbl, [ridx[lz, s16]])
        return carry

    lax.fori_loop(0, 20, p1, 0)

    pltpu.sync_copy(isc_hbm, tbl)

    def p2(z, carry):
        pltpu.sync_copy(idx_hbm.at[1, pl.ds(r0 + z * 8, 8), :], ibuf)
        for q in range(8):
            lz = z * 8 + q
            for k in range(8):
                s16 = pl.ds(16 * k, 16)
                vhacc[lz, s16] = (vhacc[lz, s16]
                                  * plsc.load_gather(tbl, [ibuf[q, s16]]))
        return carry

    lax.fori_loop(0, 20, p2, 0)
    pltpu.sync_copy(vhacc, vh_hbm.at[pl.ds(r0, 160), :])

    pltpu.sync_copy(inv_hbm, tbl)

    def p3(z, carry):
        for q in range(8):
            lz = z * 8 + q
            for k in range(8):
                s16 = pl.ds(16 * k, 16)
                obuf[q, s16] = plsc.load_gather(tbl, [ridx[lz, s16]])
        pltpu.sync_copy(obuf, vals_hbm.at[pl.ds(r0 + z * 8, 8), :])
        return carry

    lax.fori_loop(0, 20, p3, 0)


def _k3_call(idx2, isr_t, isc_t, inv_t):
    out = jax.ShapeDtypeStruct((ROWS2E, 128), jnp.float32)
    kfn = pl.kernel(
        _k3_body,
        out_type=[out, out],
        mesh=plsc.VectorSubcoreMesh(**_MESH),
        compiler_params=pltpu.CompilerParams(use_tc_tiling_on_sc=False, needs_layout_passes=False),
        scratch_types=[
            pltpu.VMEM((RNP,), jnp.float32),
            pltpu.VMEM((160, 128), jnp.float32),
            pltpu.VMEM((160, 128), jnp.int32),
            pltpu.VMEM((8, 128), jnp.int32),
            pltpu.VMEM((8, 128), jnp.float32),
        ],
    )
    return kfn(idx2, isr_t, isc_t, inv_t)


def _k0_body(ne_ref, w_ref, b_ref, out_ref):
    hb = jnp.dot(ne_ref[...], w_ref[...],
                 preferred_element_type=jnp.float32) + b_ref[...]
    m = jnp.mean(hb, axis=1, keepdims=True)
    d = jnp.sqrt(jnp.sum((hb - m) * (hb - m), axis=1, keepdims=True)
                 / (H - 1))
    o = (hb - m) / d
    out_ref[...] = jnp.where(jnp.isnan(o), 0.0, o)


def _k0_call(ne, w, b):
    return pl.pallas_call(
        _k0_body,
        grid=(10,),
        in_specs=[
            pl.BlockSpec((1000, 128), lambda i: (i, 0)),
            pl.BlockSpec((128, 128), lambda i: (0, 0)),
            pl.BlockSpec((1, 128), lambda i: (0, 0)),
        ],
        out_specs=pl.BlockSpec((1000, 128), lambda i: (i, 0)),
        out_shape=jax.ShapeDtypeStruct((N, H), jnp.float32),
    )(ne, w, b)


def _lane_bcast(v16, l):
    idx = jnp.full((16,), l, jnp.int32)
    return jnp.take_along_axis(v16, idx, axis=0, mode="promise_in_bounds")


def _k4_body(meta_hbm, xs2_hbm, tv_hbm, mbuf, gx, acc, gsem):
    c = lax.axis_index("c")
    t = lax.axis_index("s")
    wid = c * NS + t
    t0 = wid * 80

    def issue(b, rr):
        pltpu.sync_copy(meta_hbm.at[rr], mbuf.at[b])
        for q in range(4):
            pltpu.async_copy(xs2_hbm.at[mbuf.at[b, q]], gx.at[b, q], gsem)

    def drain(b):
        for q in range(4):
            pltpu.make_async_copy(xs2_hbm.at[mbuf.at[b, q]], gx.at[b, q],
                                  gsem).wait()

    issue(0, t0)
    issue(1, t0 + 1)
    z16 = jnp.zeros((16,), jnp.float32)
    accs0 = (z16,) * 8

    def pair(ii, accs):
        for b in range(2):
            drain(b)
            accs = _tv_edges(b, mbuf, gx, accs)

            @pl.when(ii < 39)
            def _():
                issue(b, t0 + 2 * ii + b + 2)

        return accs

    accs = lax.fori_loop(0, 40, pair, accs0)
    for r in range(8):
        acc[r, :] = accs[r]
    pltpu.sync_copy(acc, tv_hbm.at[wid])


def _tv_edges(b, mbuf, gx, accs):
    z16 = jnp.zeros((16,), jnp.float32)

    def grp(jj, accs):
        accs = list(accs)
        base = 16 * jj
        bs = pl.ds(base, 16)
        p16 = mbuf[b, 4, bs]
        vf16 = plsc.bitcast(mbuf[b, 5, bs], jnp.float32)
        vm16 = plsc.bitcast(mbuf[b, 6, bs], jnp.float32)
        for l in range(16):
            j = base + l
            pva = gx[b, 0, j, pl.ds(0, 16)] * gx[b, 2, j, pl.ds(0, 16)]
            pvb = gx[b, 1, j, pl.ds(0, 16)] * gx[b, 3, j, pl.ds(0, 16)]
            for k in range(1, 4):
                s16 = pl.ds(16 * k, 16)
                pva = pva + gx[b, 0, j, s16] * gx[b, 2, j, s16]
                pvb = pvb + gx[b, 1, j, s16] * gx[b, 3, j, s16]
            pv = pva + pvb
            pjb = _lane_bcast(p16, l)
            vfb = _lane_bcast(vf16, l)
            vmb = _lane_bcast(vm16, l)
            for r in range(4):
                m = pjb == r
                accs[r] = accs[r] + jnp.where(m, vfb, z16) * pv
                accs[r + 4] = accs[r + 4] + jnp.where(m, vmb, z16) * pv
        return tuple(accs)

    return lax.fori_loop(0, 8, grp, tuple(accs))


def _k4_call(meta4, xs2):
    kfn = pl.kernel(
        _k4_body,
        out_type=jax.ShapeDtypeStruct((32, 8, 16), jnp.float32),
        mesh=plsc.VectorSubcoreMesh(**_MESH),
        compiler_params=pltpu.CompilerParams(use_tc_tiling_on_sc=False, needs_layout_passes=False),
        scratch_types=[
            pltpu.VMEM((2, 8, 128), jnp.int32),
            pltpu.VMEM((2, 4, 128, HH), jnp.float32),
            pltpu.VMEM((8, 16), jnp.float32),
            pltpu.SemaphoreType.DMA,
        ],
    )
    return kfn(meta4, xs2)


def _k6_body(meta_hbm, u16_hbm, xs2_hbm, zeros_hbm, xsn_hbm,
             accum, mbuf, grows, stage, wbuf, ubuf, gsem, ssem):
    c = lax.axis_index("c")
    t = lax.axis_index("s")
    pltpu.sync_copy(zeros_hbm.at[pl.ds(t * 625, 625), :],
                    accum.at[pl.ds(t * 625, 625), :])
    pltpu.sync_copy(u16_hbm, ubuf)
    plsc.subcore_barrier()
    coff = c * N
    t0 = t * 320

    def load_meta_and_gather(m, b, rr):
        pltpu.sync_copy(meta_hbm.at[c, rr], mbuf.at[m])
        pltpu.async_copy(xs2_hbm.at[mbuf.at[m, 0]], grows.at[b], gsem)

    load_meta_and_gather(0, 0, t0)
    load_meta_and_gather(1, 1, t0 + 1)

    def quad(ii, carry):
        for q in range(4):
            b = q % 2
            m = q
            i = 4 * ii + q
            r = t0 + i
            pltpu.make_async_copy(xs2_hbm.at[mbuf.at[m, 0]], grows.at[b],
                                  gsem).wait()

            @pl.when(i >= 2)
            def _():
                pltpu.make_async_copy(stage.at[b],
                                      accum.at[mbuf.at[m, 1]], ssem).wait()

            for k in range(8):
                s16 = pl.ds(16 * k, 16)
                pk = mbuf[m, 2, s16]
                vk = lax.bitcast_convert_type(mbuf[m, 3, s16], jnp.float32)
                wbuf[s16] = plsc.load_gather(ubuf, [pk]) * vk

            def grp(jj, cy):
                base = 16 * jj
                w16 = wbuf[pl.ds(base, 16)]
                for l in range(16):
                    j = base + l
                    wjb = _lane_bcast(w16, l)
                    for k in range(4):
                        s16 = pl.ds(16 * k, 16)
                        stage[b, j, s16] = grows[b, j, s16] * wjb
                return cy

            lax.fori_loop(0, 8, grp, 0)
            pltpu.async_copy(stage.at[b], accum.at[mbuf.at[m, 1]], ssem,
                             add=True)

            @pl.when(i + 2 < 320)
            def _():
                load_meta_and_gather((m + 2) % 4, b, r + 2)

        return carry

    lax.fori_loop(0, 80, quad, 0)
    for q in range(2, 4):
        pltpu.make_async_copy(stage.at[q % 2], accum.at[mbuf.at[q, 1]],
                              ssem).wait()
    plsc.subcore_barrier()

    def cblk(qq, carry):
        r0 = t * 625 + qq * 25
        g0 = coff + r0
        pltpu.sync_copy(accum.at[pl.ds(r0, 25), :], grows.at[0, pl.ds(0, 25)])
        pltpu.sync_copy(xs2_hbm.at[pl.ds(g0, 25), :],
                        grows.at[1, pl.ds(0, 25)])
        for i in range(25):
            for k in range(4):
                s16 = pl.ds(16 * k, 16)
                grows[1, i, s16] = (C1 * grows[1, i, s16]
                                    + C2 * grows[0, i, s16])
        pltpu.sync_copy(grows.at[1, pl.ds(0, 25)], xsn_hbm.at[pl.ds(g0, 25), :])
        return carry

    lax.fori_loop(0, 25, cblk, 0)


def _k6_call(meta6, u16, xs2, zk6):
    kfn = pl.kernel(
        _k6_body,
        out_type=jax.ShapeDtypeStruct((2 * N, HH), jnp.float32),
        mesh=plsc.VectorSubcoreMesh(**_MESH),
        compiler_params=pltpu.CompilerParams(use_tc_tiling_on_sc=False, needs_layout_passes=False),
        scratch_types=[
            pltpu.VMEM_SHARED((N, HH), jnp.float32),
            pltpu.VMEM((4, 4, 128), jnp.int32),
            pltpu.VMEM((2, 128, HH), jnp.float32),
            pltpu.VMEM((2, 128, HH), jnp.float32),
            pltpu.VMEM((128,), jnp.float32),
            pltpu.VMEM((16,), jnp.float32),
            pltpu.SemaphoreType.DMA,
            pltpu.SemaphoreType.DMA,
        ],
    )
    return kfn(meta6, u16, xs2, zk6)


def _k5_body(tv_ref, xs_ref, u_ref, uo_ref):
    dots = jnp.sum(tv_ref[...], axis=1)
    nsq = jnp.sum(xs_ref[...] * xs_ref[...])
    w = jnp.broadcast_to(((nsq - dots) / N)[:, None], (R8, 128))
    l1 = jnp.sum(jnp.abs(w[:, :1]))
    fi = l1 + 2.0 * LAM2 / LAM1
    u = u_ref[...]

    def inner(tt, uu):
        t_f = (tt + 1).astype(jnp.float32)
        T_t = jnp.sqrt(2.0 * math.log(R8) / (t_f * fi * fi))
        f_de = (2.0 * LAM2 / LAM1) * uu + w
        u_ta = uu * jnp.exp(-T_t * f_de)
        return u_ta / jnp.sum(u_ta[:, :1])

    uo_ref[...] = lax.fori_loop(0, 10, inner, u)


def _k5_call(tvr, xs2, u):
    return pl.pallas_call(
        _k5_body,
        out_shape=jax.ShapeDtypeStruct((R8, 128), jnp.float32),
    )(tvr, xs2, u)


def _k7_body(xa_ref, xb_ref, wa_ref, wb_ref, b_ref, o_ref):
    o_ref[...] = (jnp.dot(xa_ref[...], wa_ref[...],
                          preferred_element_type=jnp.float32)
                  + jnp.dot(xb_ref[...], wb_ref[...],
                            preferred_element_type=jnp.float32)
                  + b_ref[...])


def _k7_call(xs2, wa, wb, b):
    return pl.pallas_call(
        _k7_body,
        grid=(10,),
        in_specs=[
            pl.BlockSpec((1000, HH), lambda i: (i, 0)),
            pl.BlockSpec((1000, HH), lambda i: (i + 10, 0)),
            pl.BlockSpec((HH, 128), lambda i: (0, 0)),
            pl.BlockSpec((HH, 128), lambda i: (0, 0)),
            pl.BlockSpec((1, 128), lambda i: (0, 0)),
        ],
        out_specs=pl.BlockSpec((1000, 128), lambda i: (i, 0)),
        out_shape=jax.ShapeDtypeStruct((N, 128), jnp.float32),
    )(xs2, xs2, wa, wb, b)


def kernel(features_list, norm, node_embeddings, linear_w, linear_b, out_w,
           out_b, g, r, num_nodes, num_relations, args_dataset):
    del features_list, norm, r, num_nodes, num_relations, args_dataset
    g = g.astype(jnp.int32)
    s = g[:, 0]
    p = jnp.remainder(g[:, 1], RB)
    o = g[:, 2]
    s_all = jnp.concatenate([s, o])
    p_all = jnp.concatenate([p, p + RB])
    o_all = jnp.concatenate([o, s])
    rows = p_all * N + s_all
    colseg = p_all * N + o_all

    pad2 = PAD2E - E2
    padfill = jnp.full((pad2,), RN, jnp.int32)
    idx2 = jnp.stack([jnp.concatenate([rows, padfill]),
                      jnp.concatenate([colseg, padfill])]).reshape(
                          2, ROWS2E, 128)
    zk1 = jnp.zeros((16, 5008), jnp.float32)
    counts = _k1_call(idx2, zk1)

    isr2, isc2, inv2 = _k2_call(counts)
    zpad = jnp.zeros((128,), jnp.float32)
    isr_t = jnp.concatenate([isr2, zpad])
    isc_t = jnp.concatenate([isc2, zpad])
    inv_t = jnp.concatenate([inv2, zpad])

    vals2d, vh2d = _k3_call(idx2, isr_t, isc_t, inv_t)

    vh_flat = vh2d.reshape(PAD2E)
    vhf = vh_flat[:E]
    vhm = vh_flat[E:E2]
    padE = PADE - E
    zi = jnp.zeros((padE,), jnp.int32)
    zf = jnp.zeros((padE,), jnp.float32)

    def padi(x):
        return jnp.concatenate([x, zi])

    def padf(x):
        return lax.bitcast_convert_type(jnp.concatenate([x, zf]), jnp.int32)

    meta4 = jnp.stack([padi(s), padi(s + N), padi(o), padi(o + N), padi(p),
                       padf(vhf), padf(vhm), padi(jnp.zeros((E,), jnp.int32))],
                      axis=0)
    meta4 = meta4.reshape(8, ROWSE, 128).transpose(1, 0, 2)

    zi2 = jnp.zeros((pad2,), jnp.int32)

    def padi2(x):
        return jnp.concatenate([x, zi2])

    vals_bits = lax.bitcast_convert_type(vals2d.reshape(PAD2E), jnp.int32)
    meta6 = jnp.stack([
        jnp.stack([padi2(o_all), padi2(s_all), padi2(p_all), vals_bits]),
        jnp.stack([padi2(o_all + N), padi2(s_all), padi2(p_all), vals_bits]),
    ])
    meta6 = meta6.reshape(2, 4, ROWS2E, 128).transpose(0, 2, 1, 3)

    xn = _k0_call(node_embeddings, linear_w, linear_b.reshape(1, H))
    xs2 = jnp.concatenate([xn[:, :HH], xn[:, HH:]], axis=0)

    zk6 = jnp.zeros((N, HH), jnp.float32)
    u = jnp.full((R8, 128), 1.0 / R8, jnp.float32)
    for _ in range(2):
        tvt = _k4_call(meta4, xs2)
        tvr = tvt.transpose(1, 0, 2).reshape(R8, 512)
        u = _k5_call(tvr, xs2, u)
        u16 = jnp.concatenate([u[:, 0], jnp.zeros((8,), jnp.float32)])
        xs2 = _k6_call(meta6, u16, xs2, zk6)

    out_wp = jnp.pad(out_w, ((0, 0), (0, 112)))
    logits_full = _k7_call(xs2, out_wp[:HH], out_wp[HH:],
                           jnp.pad(out_b, (0, 112)).reshape(1, 128))
    logits = logits_full[:, :16]
    embeddings = jnp.concatenate([xs2[:N], xs2[N:]], axis=1)
    return logits, embeddings, u[:, :1]


# speedup vs baseline: 9.5183x; 1.2058x over previous
import functools
import math

import jax
import jax.numpy as jnp
from jax import lax
from jax.experimental import pallas as pl
from jax.experimental.pallas import tpu as pltpu
from jax.experimental.pallas import tpu_sc as plsc

N = 10000
E = 320000
RB = 4
R8 = 8
H = 128
HH = 64
RN = R8 * N
RNP = RN + 128
E2 = 2 * E
PAD2E = 655360
ROWS2E = PAD2E // 128
PADE = 327680
ROWSE = PADE // 128
LAM1 = 20.0
LAM2 = 30.0
C1 = 1.0 / (1.0 + LAM1)
C2 = LAM1 / (1.0 + LAM1)
NC = 2
NS = 16

_MESH = dict(core_axis_name="c", subcore_axis_name="s", num_cores=NC,
             num_subcores=NS)


def _f32(x):
    return x.astype(jnp.float32)


def _k1_body(idx_hbm, zeros_hbm, out_hbm, acc, ibuf, ones, obuf, sem):
    c = lax.axis_index("c")
    t = lax.axis_index("s")
    pltpu.sync_copy(zeros_hbm.at[t], acc.at[pl.ds(t * 5008, 5008)])
    for k in range(8):
        ones[pl.ds(16 * k, 16)] = jnp.full((16,), 1.0, jnp.float32)
    plsc.subcore_barrier()

    def blk(bi, carry):
        r0 = t * 320 + bi * 16
        pltpu.sync_copy(idx_hbm.at[c, pl.ds(r0, 16), :], ibuf)
        for b in range(16):
            pltpu.async_copy(ones, acc.at[ibuf.at[b]], sem, add=True)
        for b in range(16):
            pltpu.make_async_copy(ones, acc.at[ibuf.at[b]], sem).wait()
        return carry

    lax.fori_loop(0, 20, blk, 0)
    plsc.subcore_barrier()
    pltpu.sync_copy(acc.at[pl.ds(t * 5000, 5000)], obuf)
    pltpu.sync_copy(obuf, out_hbm.at[c, pl.ds(t * 5000, 5000)])


def _k1_call(idx2, zk1):
    kfn = pl.kernel(
        _k1_body,
        out_type=jax.ShapeDtypeStruct((2, RN), jnp.float32),
        mesh=plsc.VectorSubcoreMesh(**_MESH),
        compiler_params=pltpu.CompilerParams(use_tc_tiling_on_sc=False, needs_layout_passes=False),
        scratch_types=[
            pltpu.VMEM_SHARED((RNP,), jnp.float32),
            pltpu.VMEM((16, 128), jnp.int32),
            pltpu.VMEM((128,), jnp.float32),
            pltpu.VMEM((5000,), jnp.float32),
            pltpu.SemaphoreType.DMA,
        ],
    )
    return kfn(idx2, zk1)


def _k2_body(cnt_ref, isr_ref, isc_ref, inv_ref):
    rs = cnt_ref[0]
    dc = cnt_ref[1]
    isr_ref[...] = jnp.where(rs > 0, 1.0 / jnp.sqrt(jnp.maximum(rs, 1e-12)),
                             0.0)
    isc_ref[...] = jnp.where(dc > 0, 1.0 / jnp.sqrt(jnp.maximum(dc, 1e-12)),
                             0.0)
    inv_ref[...] = 1.0 / jnp.maximum(rs, 1.0)


def _k2_call(counts):
    out = jax.ShapeDtypeStruct((RN,), jnp.float32)
    return pl.pallas_call(
        _k2_body,
        out_shape=[out, out, out],
    )(counts)


def _k3_body(idx_hbm, isr_hbm, isc_hbm, inv_hbm, vals_hbm, vh_hbm,
             tbl, vhacc, ridx, ibuf, obuf):
    c = lax.axis_index("c")
    t = lax.axis_index("s")
    wid = c * NS + t
    r0 = wid * 160
    pltpu.sync_copy(idx_hbm.at[0, pl.ds(r0, 160), :], ridx)

    pltpu.sync_copy(isr_hbm, tbl)

    def p1(z, carry):
        for q in range(8):
            lz = z * 8 + q
            for k in range(8):
                s16 = pl.ds(16 * k, 16)
                vhacc[lz, s16] = plsc.load_gather(tbl, [ridx[lz, s16]])
        return carry

    lax.fori_loop(0, 20, p1, 0)

    pltpu.sync_copy(isc_hbm, tbl)

    def p2(z, carry):
        pltpu.sync_copy(idx_hbm.at[1, pl.ds(r0 + z * 8, 8), :], ibuf)
        for q in range(8):
            lz = z * 8 + q
            for k in range(8):
                s16 = pl.ds(16 * k, 16)
                vhacc[lz, s16] = (vhacc[lz, s16]
                                  * plsc.load_gather(tbl, [ibuf[q, s16]]))
        return carry

    lax.fori_loop(0, 20, p2, 0)
    pltpu.sync_copy(vhacc, vh_hbm.at[pl.ds(r0, 160), :])

    pltpu.sync_copy(inv_hbm, tbl)

    def p3(z, carry):
        for q in range(8):
            lz = z * 8 + q
            for k in range(8):
                s16 = pl.ds(16 * k, 16)
                obuf[q, s16] = plsc.load_gather(tbl, [ridx[lz, s16]])
        pltpu.sync_copy(obuf, vals_hbm.at[pl.ds(r0 + z * 8, 8), :])
        return carry

    lax.fori_loop(0, 20, p3, 0)


def _k3_call(idx2, isr_t, isc_t, inv_t):
    out = jax.ShapeDtypeStruct((ROWS2E, 128), jnp.float32)
    kfn = pl.kernel(
        _k3_body,
        out_type=[out, out],
        mesh=plsc.VectorSubcoreMesh(**_MESH),
        compiler_params=pltpu.CompilerParams(use_tc_tiling_on_sc=False, needs_layout_passes=False),
        scratch_types=[
            pltpu.VMEM((RNP,), jnp.float32),
            pltpu.VMEM((160, 128), jnp.float32),
            pltpu.VMEM((160, 128), jnp.int32),
            pltpu.VMEM((8, 128), jnp.int32),
            pltpu.VMEM((8, 128), jnp.float32),
        ],
    )
    return kfn(idx2, isr_t, isc_t, inv_t)


def _k0_body(ne_ref, w_ref, b_ref, out_ref):
    hb = jnp.dot(ne_ref[...], w_ref[...],
                 preferred_element_type=jnp.float32) + b_ref[...]
    m = jnp.mean(hb, axis=1, keepdims=True)
    d = jnp.sqrt(jnp.sum((hb - m) * (hb - m), axis=1, keepdims=True)
                 / (H - 1))
    o = (hb - m) / d
    out_ref[...] = jnp.where(jnp.isnan(o), 0.0, o)


def _k0_call(ne, w, b):
    return pl.pallas_call(
        _k0_body,
        grid=(10,),
        in_specs=[
            pl.BlockSpec((1000, 128), lambda i: (i, 0)),
            pl.BlockSpec((128, 128), lambda i: (0, 0)),
            pl.BlockSpec((1, 128), lambda i: (0, 0)),
        ],
        out_specs=pl.BlockSpec((1000, 128), lambda i: (i, 0)),
        out_shape=jax.ShapeDtypeStruct((N, H), jnp.float32),
    )(ne, w, b)


def _lane_bcast(v16, l):
    idx = jnp.full((16,), l, jnp.int32)
    return jnp.take_along_axis(v16, idx, axis=0, mode="promise_in_bounds")


def _k4_body(meta_hbm, xs2_hbm, tv_hbm, xtab, mbuf, gx, acc, gsem):
    c = lax.axis_index("c")
    t = lax.axis_index("s")
    wid = c * NS + t
    t0 = t * 160
    pltpu.sync_copy(xs2_hbm.at[pl.ds(c * N + t * 625, 625), :],
                    xtab.at[pl.ds(t * 625, 625), :])
    plsc.subcore_barrier()

    def issue(b, rr):
        pltpu.sync_copy(meta_hbm.at[rr], mbuf.at[b])
        for q in range(2):
            pltpu.async_copy(xtab.at[mbuf.at[b, q]], gx.at[b, q], gsem)

    def drain(b):
        for q in range(2):
            pltpu.make_async_copy(xtab.at[mbuf.at[b, q]], gx.at[b, q],
                                  gsem).wait()

    issue(0, t0)
    issue(1, t0 + 1)
    z16 = jnp.zeros((16,), jnp.float32)
    accs0 = (z16,) * 8

    def pair(ii, accs):
        for b in range(2):
            drain(b)
            accs = _tv_edges(b, mbuf, gx, accs)

            @pl.when(ii < 79)
            def _():
                issue(b, t0 + 2 * ii + b + 2)

        return accs

    accs = lax.fori_loop(0, 80, pair, accs0)
    for r in range(8):
        acc[r, :] = accs[r]
    pltpu.sync_copy(acc, tv_hbm.at[wid])


def _tv_edges(b, mbuf, gx, accs):
    z16 = jnp.zeros((16,), jnp.float32)

    def grp(jj, accs):
        accs = list(accs)
        base = 16 * jj
        bs = pl.ds(base, 16)
        p16 = mbuf[b, 2, bs]
        vf16 = plsc.bitcast(mbuf[b, 3, bs], jnp.float32)
        vm16 = plsc.bitcast(mbuf[b, 4, bs], jnp.float32)
        for l in range(16):
            j = base + l
            pva = gx[b, 0, j, pl.ds(0, 16)] * gx[b, 1, j, pl.ds(0, 16)]
            pvb = gx[b, 0, j, pl.ds(16, 16)] * gx[b, 1, j, pl.ds(16, 16)]
            for k in range(2, 4, 2):
                pva = pva + (gx[b, 0, j, pl.ds(16 * k, 16)]
                             * gx[b, 1, j, pl.ds(16 * k, 16)])
                pvb = pvb + (gx[b, 0, j, pl.ds(16 * k + 16, 16)]
                             * gx[b, 1, j, pl.ds(16 * k + 16, 16)])
            pv = pva + pvb
            pjb = _lane_bcast(p16, l)
            vfb = _lane_bcast(vf16, l)
            vmb = _lane_bcast(vm16, l)
            for r in range(4):
                m = pjb == r
                accs[r] = accs[r] + jnp.where(m, vfb, z16) * pv
                accs[r + 4] = accs[r + 4] + jnp.where(m, vmb, z16) * pv
        return tuple(accs)

    return lax.fori_loop(0, 8, grp, tuple(accs))


def _k4_call(meta4, xs2):
    kfn = pl.kernel(
        _k4_body,
        out_type=jax.ShapeDtypeStruct((32, 8, 16), jnp.float32),
        mesh=plsc.VectorSubcoreMesh(**_MESH),
        compiler_params=pltpu.CompilerParams(use_tc_tiling_on_sc=False, needs_layout_passes=False),
        scratch_types=[
            pltpu.VMEM_SHARED((N, HH), jnp.float32),
            pltpu.VMEM((2, 8, 128), jnp.int32),
            pltpu.VMEM((2, 2, 128, HH), jnp.float32),
            pltpu.VMEM((8, 16), jnp.float32),
            pltpu.SemaphoreType.DMA,
        ],
    )
    return kfn(meta4, xs2)


def _k6_body(meta_hbm, u16_hbm, xs2_hbm, zeros_hbm, xsn_hbm,
             accum, xtab, mbuf, grows, stage, wbuf, ubuf, gsem, ssem):
    c = lax.axis_index("c")
    t = lax.axis_index("s")
    coff = c * N
    pltpu.sync_copy(zeros_hbm.at[pl.ds(t * 625, 625), :],
                    accum.at[pl.ds(t * 625, 625), :])
    pltpu.sync_copy(xs2_hbm.at[pl.ds(coff + t * 625, 625), :],
                    xtab.at[pl.ds(t * 625, 625), :])
    pltpu.sync_copy(u16_hbm, ubuf)
    plsc.subcore_barrier()
    t0 = t * 320

    def load_meta_and_gather(m, b, rr):
        pltpu.sync_copy(meta_hbm.at[rr], mbuf.at[m])
        pltpu.async_copy(xtab.at[mbuf.at[m, 0]], grows.at[b], gsem)

    load_meta_and_gather(0, 0, t0)
    load_meta_and_gather(1, 1, t0 + 1)

    def quad(ii, carry):
        for q in range(4):
            b = q % 2
            m = q
            i = 4 * ii + q
            r = t0 + i
            pltpu.make_async_copy(xtab.at[mbuf.at[m, 0]], grows.at[b],
                                  gsem).wait()

            @pl.when(i >= 2)
            def _():
                pltpu.make_async_copy(stage.at[b],
                                      accum.at[mbuf.at[m, 1]], ssem).wait()

            for k in range(8):
                s16 = pl.ds(16 * k, 16)
                pk = mbuf[m, 2, s16]
                vk = lax.bitcast_convert_type(mbuf[m, 3, s16], jnp.float32)
                wbuf[s16] = plsc.load_gather(ubuf, [pk]) * vk

            def grp(jj, cy):
                base = 16 * jj
                w16 = wbuf[pl.ds(base, 16)]
                for l in range(16):
                    j = base + l
                    wjb = _lane_bcast(w16, l)
                    for k in range(4):
                        s16 = pl.ds(16 * k, 16)
                        stage[b, j, s16] = grows[b, j, s16] * wjb
                return cy

            lax.fori_loop(0, 8, grp, 0)
            pltpu.async_copy(stage.at[b], accum.at[mbuf.at[m, 1]], ssem,
                             add=True)

            @pl.when(i + 2 < 320)
            def _():
                load_meta_and_gather((m + 2) % 4, b, r + 2)

        return carry

    lax.fori_loop(0, 80, quad, 0)
    for q in range(2, 4):
        pltpu.make_async_copy(stage.at[q % 2], accum.at[mbuf.at[q, 1]],
                              ssem).wait()
    plsc.subcore_barrier()

    def cblk(qq, carry):
        r0 = t * 625 + qq * 25
        g0 = coff + r0
        pltpu.sync_copy(accum.at[pl.ds(r0, 25), :], grows.at[0, pl.ds(0, 25)])
        pltpu.sync_copy(xtab.at[pl.ds(r0, 25), :],
                        grows.at[1, pl.ds(0, 25)])
        for i in range(25):
            for k in range(4):
                s16 = pl.ds(16 * k, 16)
                grows[1, i, s16] = (C1 * grows[1, i, s16]
                                    + C2 * grows[0, i, s16])
        pltpu.sync_copy(grows.at[1, pl.ds(0, 25)], xsn_hbm.at[pl.ds(g0, 25), :])
        return carry

    lax.fori_loop(0, 25, cblk, 0)


def _k6_call(meta6, u16, xs2, zk6):
    kfn = pl.kernel(
        _k6_body,
        out_type=jax.ShapeDtypeStruct((2 * N, HH), jnp.float32),
        mesh=plsc.VectorSubcoreMesh(**_MESH),
        compiler_params=pltpu.CompilerParams(use_tc_tiling_on_sc=False, needs_layout_passes=False),
        scratch_types=[
            pltpu.VMEM_SHARED((N, HH), jnp.float32),
            pltpu.VMEM_SHARED((N, HH), jnp.float32),
            pltpu.VMEM((4, 4, 128), jnp.int32),
            pltpu.VMEM((2, 128, HH), jnp.float32),
            pltpu.VMEM((2, 128, HH), jnp.float32),
            pltpu.VMEM((128,), jnp.float32),
            pltpu.VMEM((16,), jnp.float32),
            pltpu.SemaphoreType.DMA,
            pltpu.SemaphoreType.DMA,
        ],
    )
    return kfn(meta6, u16, xs2, zk6)


def _k5_body(tv_ref, xs_ref, u_ref, uo_ref):
    dots = jnp.sum(tv_ref[...], axis=1)
    nsq = jnp.sum(xs_ref[...] * xs_ref[...])
    w = jnp.broadcast_to(((nsq - dots) / N)[:, None], (R8, 128))
    l1 = jnp.sum(jnp.abs(w[:, :1]))
    fi = l1 + 2.0 * LAM2 / LAM1
    u = u_ref[...]

    def inner(tt, uu):
        t_f = (tt + 1).astype(jnp.float32)
        T_t = jnp.sqrt(2.0 * math.log(R8) / (t_f * fi * fi))
        f_de = (2.0 * LAM2 / LAM1) * uu + w
        u_ta = uu * jnp.exp(-T_t * f_de)
        return u_ta / jnp.sum(u_ta[:, :1])

    uo_ref[...] = lax.fori_loop(0, 10, inner, u)


def _k5_call(tvr, xs2, u):
    return pl.pallas_call(
        _k5_body,
        out_shape=jax.ShapeDtypeStruct((R8, 128), jnp.float32),
    )(tvr, xs2, u)


def _k7_body(xa_ref, xb_ref, wa_ref, wb_ref, b_ref, o_ref):
    o_ref[...] = (jnp.dot(xa_ref[...], wa_ref[...],
                          preferred_element_type=jnp.float32)
                  + jnp.dot(xb_ref[...], wb_ref[...],
                            preferred_element_type=jnp.float32)
                  + b_ref[...])


def _k7_call(xs2, wa, wb, b):
    return pl.pallas_call(
        _k7_body,
        grid=(10,),
        in_specs=[
            pl.BlockSpec((1000, HH), lambda i: (i, 0)),
            pl.BlockSpec((1000, HH), lambda i: (i + 10, 0)),
            pl.BlockSpec((HH, 128), lambda i: (0, 0)),
            pl.BlockSpec((HH, 128), lambda i: (0, 0)),
            pl.BlockSpec((1, 128), lambda i: (0, 0)),
        ],
        out_specs=pl.BlockSpec((1000, 128), lambda i: (i, 0)),
        out_shape=jax.ShapeDtypeStruct((N, 128), jnp.float32),
    )(xs2, xs2, wa, wb, b)


def kernel(features_list, norm, node_embeddings, linear_w, linear_b, out_w,
           out_b, g, r, num_nodes, num_relations, args_dataset):
    del features_list, norm, r, num_nodes, num_relations, args_dataset
    g = g.astype(jnp.int32)
    s = g[:, 0]
    p = jnp.remainder(g[:, 1], RB)
    o = g[:, 2]
    s_all = jnp.concatenate([s, o])
    p_all = jnp.concatenate([p, p + RB])
    o_all = jnp.concatenate([o, s])
    rows = p_all * N + s_all
    colseg = p_all * N + o_all

    pad2 = PAD2E - E2
    padfill = jnp.full((pad2,), RN, jnp.int32)
    idx2 = jnp.stack([jnp.concatenate([rows, padfill]),
                      jnp.concatenate([colseg, padfill])]).reshape(
                          2, ROWS2E, 128)
    zk1 = jnp.zeros((16, 5008), jnp.float32)
    counts = _k1_call(idx2, zk1)

    isr2, isc2, inv2 = _k2_call(counts)
    zpad = jnp.zeros((128,), jnp.float32)
    isr_t = jnp.concatenate([isr2, zpad])
    isc_t = jnp.concatenate([isc2, zpad])
    inv_t = jnp.concatenate([inv2, zpad])

    vals2d, vh2d = _k3_call(idx2, isr_t, isc_t, inv_t)

    vh_flat = vh2d.reshape(PAD2E)
    vhf = vh_flat[:E]
    vhm = vh_flat[E:E2]
    padE = PADE - E
    zi = jnp.zeros((padE,), jnp.int32)
    zf = jnp.zeros((padE,), jnp.float32)

    def padi(x):
        return jnp.concatenate([x, zi])

    def padf(x):
        return lax.bitcast_convert_type(jnp.concatenate([x, zf]), jnp.int32)

    zrow = padi(jnp.zeros((E,), jnp.int32))
    meta4 = jnp.stack([padi(s), padi(o), padi(p),
                       padf(vhf), padf(vhm), zrow, zrow, zrow], axis=0)
    meta4 = meta4.reshape(8, ROWSE, 128).transpose(1, 0, 2)

    zi2 = jnp.zeros((pad2,), jnp.int32)

    def padi2(x):
        return jnp.concatenate([x, zi2])

    vals_bits = lax.bitcast_convert_type(vals2d.reshape(PAD2E), jnp.int32)
    meta6 = jnp.stack([padi2(o_all), padi2(s_all), padi2(p_all), vals_bits])
    meta6 = meta6.reshape(4, ROWS2E, 128).transpose(1, 0, 2)

    xn = _k0_call(node_embeddings, linear_w, linear_b.reshape(1, H))
    xs2 = jnp.concatenate([xn[:, :HH], xn[:, HH:]], axis=0)

    zk6 = jnp.zeros((N, HH), jnp.float32)
    u = jnp.full((R8, 128), 1.0 / R8, jnp.float32)
    for _ in range(2):
        tvt = _k4_call(meta4, xs2)
        tvr = tvt.transpose(1, 0, 2).reshape(R8, 512)
        u = _k5_call(tvr, xs2, u)
        u16 = jnp.concatenate([u[:, 0], jnp.zeros((8,), jnp.float32)])
        xs2 = _k6_call(meta6, u16, xs2, zk6)

    out_wp = jnp.pad(out_w, ((0, 0), (0, 112)))
    logits_full = _k7_call(xs2, out_wp[:HH], out_wp[HH:],
                           jnp.pad(out_b, (0, 112)).reshape(1, 128))
    logits = logits_full[:, :16]
    embeddings = jnp.concatenate([xs2[:N], xs2[N:]], axis=1)
    return logits, embeddings, u[:, :1]
